# baseline (speedup 1.0000x reference)
import functools

import jax
import jax.numpy as jnp
from jax import lax
from jax.experimental import pallas as pl
from jax.experimental.pallas import tpu as pltpu
from jax.experimental.pallas import tpu_sc as plsc

_BATCH = 4096
_SEQ = 200
_D = 64
_HALF = 100
_NCLS = 2


@functools.cache
def _build_pool():
    info = plsc.get_sparse_core_info()
    nc, ns = info.num_cores, info.num_subcores
    nw = nc * ns
    bpw = _BATCH // nw
    mesh = plsc.VectorSubcoreMesh(core_axis_name="c", subcore_axis_name="s")

    @functools.partial(
        pl.kernel,
        mesh=mesh,
        compiler_params=pltpu.CompilerParams(
            use_tc_tiling_on_sc=False, needs_layout_passes=False
        ),
        out_type=jax.ShapeDtypeStruct((_BATCH, _D), jnp.float32),
        scratch_types=[
            pltpu.VMEM((2 * bpw, _HALF), jnp.int32),
            pltpu.VMEM((_SEQ, _D), jnp.bfloat16),
            pltpu.VMEM((_SEQ, _D), jnp.bfloat16),
            pltpu.VMEM((bpw, _D), jnp.float32),
            pltpu.SemaphoreType.DMA,
            pltpu.SemaphoreType.DMA,
        ],
    )
    def pool(ids_hbm, table_hbm, x_hbm, idx_v, rows_a, rows_b, out_v, sem_a, sem_b):
        wid = lax.axis_index("s") * nc + lax.axis_index("c")
        pltpu.sync_copy(ids_hbm.at[wid], idx_v)
        scale = jnp.float32(1.0 / _SEQ)
        lane = lax.iota(jnp.int32, 16)

        def start(buf, sem, j0):
            pltpu.async_copy(table_hbm.at[idx_v.at[j0]], buf.at[pl.ds(0, _HALF)], sem)
            pltpu.async_copy(
                table_hbm.at[idx_v.at[j0 + 1]], buf.at[pl.ds(_HALF, _HALF)], sem
            )

        def wait(buf, sem, j0):
            pltpu.make_async_copy(
                table_hbm.at[idx_v.at[j0]], buf.at[pl.ds(0, _HALF)], sem
            ).wait()
            pltpu.make_async_copy(
                table_hbm.at[idx_v.at[j0 + 1]], buf.at[pl.ds(_HALF, _HALF)], sem
            ).wait()

        def accum(buf, row):
            def tbody(t, accs):
                new = []
                for c in range(2):
                    lo = buf[t, pl.ds(32 * c, 32)]
                    hi = buf[t + _HALF, pl.ds(32 * c, 32)]
                    elo, olo = plsc.unpack(lo, format=plsc.PackFormat.INTERLEAVED)
                    ehi, ohi = plsc.unpack(hi, format=plsc.PackFormat.INTERLEAVED)
                    new.append((accs[c][0] + elo + ehi, accs[c][1] + olo + ohi))
                return tuple(new)

            zero = jnp.zeros((16,), jnp.float32)
            accs = lax.fori_loop(
                0, _HALF, tbody, ((zero, zero), (zero, zero)), unroll=4
            )
            row_ref = out_v.at[row]
            for c in range(2):
                cols = 32 * c + 2 * lane
                plsc.store_scatter(row_ref, [cols], accs[c][0] * scale)
                plsc.store_scatter(row_ref, [cols + 1], accs[c][1] * scale)

        start(rows_a, sem_a, 0)

        def body(i, carry):
            j_a = 4 * i
            start(rows_b, sem_b, j_a + 2)
            wait(rows_a, sem_a, j_a)
            accum(rows_a, 2 * i)
            j_next = jnp.minimum(j_a + 4, 2 * bpw - 2)
            start(rows_a, sem_a, j_next)
            wait(rows_b, sem_b, j_a + 2)
            accum(rows_b, 2 * i + 1)
            return carry

        lax.fori_loop(0, bpw // 2, body, 0)
        wait(rows_a, sem_a, 2 * bpw - 2)
        pltpu.sync_copy(out_v, x_hbm.at[pl.ds(wid * bpw, bpw)])

    return pool, nw, bpw


def _linear_body(x_ref, w_ref, b_ref, o_ref):
    o_ref[...] = (
        jnp.dot(x_ref[...], w_ref[...], preferred_element_type=jnp.float32)
        + b_ref[...]
    )


def _linear(x, w, b):
    return pl.pallas_call(
        _linear_body,
        out_shape=jax.ShapeDtypeStruct((_BATCH, _NCLS), jnp.float32),
    )(x, w, b.reshape(1, _NCLS))


def kernel(input_ids, embedding, W, b):
    pool, nw, bpw = _build_pool()
    ids = input_ids.astype(jnp.int32).reshape(nw, 2 * bpw, _HALF)
    x = pool(ids, embedding.astype(jnp.bfloat16))
    logits = _linear(x, W, b)
    return (logits, x)


# speedup vs baseline: 1.0294x; 1.0294x over previous
import functools

import jax
import jax.numpy as jnp
from jax import lax
from jax.experimental import pallas as pl
from jax.experimental.pallas import tpu as pltpu
from jax.experimental.pallas import tpu_sc as plsc

_BATCH = 4096
_SEQ = 200
_D = 64
_HALF = 100
_NCLS = 2


@functools.cache
def _build_pool():
    info = plsc.get_sparse_core_info()
    nc, ns = info.num_cores, info.num_subcores
    nw = nc * ns
    bpw = _BATCH // nw
    mesh = plsc.VectorSubcoreMesh(core_axis_name="c", subcore_axis_name="s")

    @functools.partial(
        pl.kernel,
        mesh=mesh,
        compiler_params=pltpu.CompilerParams(
            use_tc_tiling_on_sc=False, needs_layout_passes=False
        ),
        out_type=jax.ShapeDtypeStruct((_BATCH, _D), jnp.float32),
        scratch_types=[
            pltpu.VMEM((bpw, _SEQ), jnp.int32),
            pltpu.VMEM((_SEQ, _D), jnp.bfloat16),
            pltpu.VMEM((_SEQ, _D), jnp.bfloat16),
            pltpu.VMEM((bpw, _D), jnp.float32),
            pltpu.SemaphoreType.DMA,
            pltpu.SemaphoreType.DMA,
        ],
    )
    def pool(ids_hbm, table_hbm, x_hbm, idx_v, rows_a, rows_b, out_v, sem_a, sem_b):
        wid = lax.axis_index("s") * nc + lax.axis_index("c")
        pltpu.sync_copy(ids_hbm.at[pl.ds(wid * bpw, bpw)], idx_v)
        scale = jnp.float32(1.0 / _SEQ)
        lane = lax.iota(jnp.int32, 16)
        c0, c1 = 104, _SEQ - 104

        def start(buf, sem, row):
            pltpu.async_copy(
                table_hbm.at[idx_v.at[row, pl.ds(0, c0)]], buf.at[pl.ds(0, c0)], sem
            )
            pltpu.async_copy(
                table_hbm.at[idx_v.at[row, pl.ds(c0, c1)]], buf.at[pl.ds(c0, c1)], sem
            )

        def wait(buf, sem, row):
            pltpu.make_async_copy(
                table_hbm.at[idx_v.at[row, pl.ds(0, c0)]], buf.at[pl.ds(0, c0)], sem
            ).wait()
            pltpu.make_async_copy(
                table_hbm.at[idx_v.at[row, pl.ds(c0, c1)]], buf.at[pl.ds(c0, c1)], sem
            ).wait()

        def accum(buf, row):
            def tbody(t, accs):
                new = []
                for c in range(2):
                    lo = buf[t, pl.ds(32 * c, 32)]
                    hi = buf[t + _HALF, pl.ds(32 * c, 32)]
                    elo, olo = plsc.unpack(lo, format=plsc.PackFormat.INTERLEAVED)
                    ehi, ohi = plsc.unpack(hi, format=plsc.PackFormat.INTERLEAVED)
                    new.append((accs[c][0] + elo + ehi, accs[c][1] + olo + ohi))
                return tuple(new)

            zero = jnp.zeros((16,), jnp.float32)
            accs = lax.fori_loop(
                0, _HALF, tbody, ((zero, zero), (zero, zero)), unroll=4
            )
            row_ref = out_v.at[row]
            for c in range(2):
                cols = 32 * c + 2 * lane
                plsc.store_scatter(row_ref, [cols], accs[c][0] * scale)
                plsc.store_scatter(row_ref, [cols + 1], accs[c][1] * scale)

        start(rows_a, sem_a, 0)

        def body(i, carry):
            r = 2 * i
            start(rows_b, sem_b, r + 1)
            wait(rows_a, sem_a, r)
            accum(rows_a, r)
            start(rows_a, sem_a, jnp.minimum(r + 2, bpw - 1))
            wait(rows_b, sem_b, r + 1)
            accum(rows_b, r + 1)
            return carry

        lax.fori_loop(0, bpw // 2, body, 0)
        wait(rows_a, sem_a, bpw - 1)
        pltpu.sync_copy(out_v, x_hbm.at[pl.ds(wid * bpw, bpw)])

    return pool, nw, bpw


def _linear_body(x_ref, w_ref, b_ref, o_ref):
    o_ref[...] = (
        jnp.dot(x_ref[...], w_ref[...], preferred_element_type=jnp.float32)
        + b_ref[...]
    )


def _linear(x, w, b):
    return pl.pallas_call(
        _linear_body,
        out_shape=jax.ShapeDtypeStruct((_BATCH, _NCLS), jnp.float32),
    )(x, w, b.reshape(1, _NCLS))


def kernel(input_ids, embedding, W, b):
    pool, nw, bpw = _build_pool()
    x = pool(input_ids.astype(jnp.int32), embedding.astype(jnp.bfloat16))
    logits = _linear(x, W, b)
    return (logits, x)


# speedup vs baseline: 1.0651x; 1.0346x over previous
import functools

import jax
import jax.numpy as jnp
from jax import lax
from jax.experimental import pallas as pl
from jax.experimental.pallas import tpu as pltpu
from jax.experimental.pallas import tpu_sc as plsc

_BATCH = 4096
_SEQ = 200
_D = 64
_NCLS = 2
_VOCAB = 50257
_CHUNK = 400


@functools.cache
def _build():
    info = plsc.get_sparse_core_info()
    nc, ns = info.num_cores, info.num_subcores
    nw = nc * ns
    bpw = _BATCH // nw
    span = -(-_VOCAB // nw)
    nchunk = -(-span // _CHUNK)
    mesh = plsc.VectorSubcoreMesh(core_axis_name="c", subcore_axis_name="s")
    params = pltpu.CompilerParams(
        use_tc_tiling_on_sc=False, needs_layout_passes=False
    )

    @functools.partial(
        pl.kernel,
        mesh=mesh,
        compiler_params=params,
        out_type=jax.ShapeDtypeStruct((nw * span, _D), jnp.bfloat16),
        scratch_types=[
            pltpu.VMEM((_CHUNK, _D), jnp.float32),
            pltpu.VMEM((_CHUNK, _D), jnp.bfloat16),
        ],
    )
    def convert(table_hbm, out_hbm, in_v, out_v):
        wid = lax.axis_index("s") * nc + lax.axis_index("c")
        sw = wid * span

        def do_chunk(k, carry):
            start = jnp.minimum(sw + k * _CHUNK, _VOCAB - _CHUNK)
            pltpu.sync_copy(table_hbm.at[pl.ds(start, _CHUNK)], in_v)

            def row(r, c2):
                for c in range(2):
                    a = in_v[r, pl.ds(32 * c, 16)]
                    b = in_v[r, pl.ds(32 * c + 16, 16)]
                    out_v[r, pl.ds(32 * c, 32)] = plsc.pack(
                        a, b, format=plsc.PackFormat.INTERLEAVED
                    )
                return c2

            lax.fori_loop(0, _CHUNK, row, 0, unroll=4)
            pltpu.sync_copy(out_v, out_hbm.at[pl.ds(start, _CHUNK)])
            return carry

        lax.fori_loop(0, nchunk, do_chunk, 0)

    @functools.partial(
        pl.kernel,
        mesh=mesh,
        compiler_params=params,
        out_type=jax.ShapeDtypeStruct((_BATCH, _D), jnp.float32),
        scratch_types=[
            pltpu.VMEM((bpw, _SEQ), jnp.int32),
            pltpu.VMEM((_SEQ, _D), jnp.bfloat16),
            pltpu.VMEM((_SEQ, _D), jnp.bfloat16),
            pltpu.VMEM((bpw, _D), jnp.float32),
            pltpu.SemaphoreType.DMA,
            pltpu.SemaphoreType.DMA,
        ],
    )
    def pool(ids_hbm, table_hbm, x_hbm, idx_v, rows_a, rows_b, out_v, sem_a, sem_b):
        wid = lax.axis_index("s") * nc + lax.axis_index("c")
        pltpu.sync_copy(ids_hbm.at[pl.ds(wid * bpw, bpw)], idx_v)
        scale = jnp.float32(1.0 / _SEQ)
        c0, c1 = 104, _SEQ - 104

        def start(buf, sem, row):
            pltpu.async_copy(
                table_hbm.at[idx_v.at[row, pl.ds(0, c0)]], buf.at[pl.ds(0, c0)], sem
            )
            pltpu.async_copy(
                table_hbm.at[idx_v.at[row, pl.ds(c0, c1)]], buf.at[pl.ds(c0, c1)], sem
            )

        def wait(buf, sem, row):
            pltpu.make_async_copy(
                table_hbm.at[idx_v.at[row, pl.ds(0, c0)]], buf.at[pl.ds(0, c0)], sem
            ).wait()
            pltpu.make_async_copy(
                table_hbm.at[idx_v.at[row, pl.ds(c0, c1)]], buf.at[pl.ds(c0, c1)], sem
            ).wait()

        def accum(buf, row):
            def tbody(t, accs):
                new = []
                for c in range(2):
                    lo = buf[t, pl.ds(32 * c, 32)]
                    hi = buf[t + 100, pl.ds(32 * c, 32)]
                    alo, blo = plsc.unpack(lo, format=plsc.PackFormat.INTERLEAVED)
                    ahi, bhi = plsc.unpack(hi, format=plsc.PackFormat.INTERLEAVED)
                    new.append((accs[c][0] + alo + ahi, accs[c][1] + blo + bhi))
                return tuple(new)

            zero = jnp.zeros((16,), jnp.float32)
            accs = lax.fori_loop(
                0, _SEQ // 2, tbody, ((zero, zero), (zero, zero)), unroll=4
            )
            for c in range(2):
                for h in range(2):
                    out_v[row, pl.ds(32 * c + 16 * h, 16)] = accs[c][h] * scale

        start(rows_a, sem_a, 0)

        def body(i, carry):
            r = 2 * i
            start(rows_b, sem_b, r + 1)
            wait(rows_a, sem_a, r)
            accum(rows_a, r)
            start(rows_a, sem_a, jnp.minimum(r + 2, bpw - 1))
            wait(rows_b, sem_b, r + 1)
            accum(rows_b, r + 1)
            return carry

        lax.fori_loop(0, bpw // 2, body, 0)
        wait(rows_a, sem_a, bpw - 1)
        pltpu.sync_copy(out_v, x_hbm.at[pl.ds(wid * bpw, bpw)])

    return convert, pool


def _linear_body(x_ref, w_ref, b_ref, o_ref):
    o_ref[...] = (
        jnp.dot(x_ref[...], w_ref[...], preferred_element_type=jnp.float32)
        + b_ref[...]
    )


def _linear(x, w, b):
    return pl.pallas_call(
        _linear_body,
        out_shape=jax.ShapeDtypeStruct((_BATCH, _NCLS), jnp.float32),
    )(x, w, b.reshape(1, _NCLS))


def kernel(input_ids, embedding, W, b):
    convert, pool = _build()
    table_bf16 = convert(embedding)
    x = pool(input_ids.astype(jnp.int32), table_bf16)
    logits = _linear(x, W, b)
    return (logits, x)


# speedup vs baseline: 1.3727x; 1.2889x over previous
import functools

import jax
import jax.numpy as jnp
from jax import lax
from jax.experimental import pallas as pl
from jax.experimental.pallas import tpu as pltpu
from jax.experimental.pallas import tpu_sc as plsc

_BATCH = 4096
_SEQ = 200
_D = 64
_NCLS = 2
_VOCAB = 50257
_CHUNK = 400


@functools.cache
def _build():
    info = plsc.get_sparse_core_info()
    nc, ns = info.num_cores, info.num_subcores
    nw = nc * ns
    bpw = _BATCH // nw
    span = -(-_VOCAB // nw)
    nchunk = -(-span // _CHUNK)
    mesh = plsc.VectorSubcoreMesh(core_axis_name="c", subcore_axis_name="s")
    params = pltpu.CompilerParams(
        use_tc_tiling_on_sc=False, needs_layout_passes=False
    )

    @functools.partial(
        pl.kernel,
        mesh=mesh,
        compiler_params=params,
        out_type=jax.ShapeDtypeStruct((nw * span, _D), jnp.bfloat16),
        scratch_types=[
            pltpu.VMEM((_CHUNK, _D), jnp.float32),
            pltpu.VMEM((_CHUNK, _D), jnp.float32),
            pltpu.VMEM((_CHUNK, _D), jnp.bfloat16),
            pltpu.VMEM((_CHUNK, _D), jnp.bfloat16),
            pltpu.SemaphoreType.DMA,
            pltpu.SemaphoreType.DMA,
        ],
    )
    def convert(table_hbm, out_hbm, in_v0, in_v1, out_v0, out_v1, semi, semo):
        wid = lax.axis_index("s") * nc + lax.axis_index("c")
        sw = wid * span
        starts = [jnp.minimum(sw + k * _CHUNK, _VOCAB - _CHUNK) for k in range(nchunk)]
        inb, outb = [in_v0, in_v1], [out_v0, out_v1]

        pltpu.async_copy(table_hbm.at[pl.ds(starts[0], _CHUNK)], inb[0], semi)
        for k in range(nchunk):
            b = k % 2
            pltpu.make_async_copy(
                table_hbm.at[pl.ds(starts[k], _CHUNK)], inb[b], semi
            ).wait()
            if k + 1 < nchunk:
                pltpu.async_copy(
                    table_hbm.at[pl.ds(starts[k + 1], _CHUNK)], inb[1 - b], semi
                )
            if k >= 2:
                pltpu.make_async_copy(
                    outb[b], out_hbm.at[pl.ds(starts[k - 2], _CHUNK)], semo
                ).wait()

            def row(r, carry, b=b):
                for c in range(2):
                    a = inb[b][r, pl.ds(32 * c, 16)]
                    z = inb[b][r, pl.ds(32 * c + 16, 16)]
                    outb[b][r, pl.ds(32 * c, 32)] = plsc.pack(
                        a, z, format=plsc.PackFormat.INTERLEAVED
                    )
                return carry

            lax.fori_loop(0, _CHUNK, row, 0, unroll=4)
            pltpu.async_copy(outb[b], out_hbm.at[pl.ds(starts[k], _CHUNK)], semo)
        for k in (nchunk - 2, nchunk - 1):
            pltpu.make_async_copy(
                outb[k % 2], out_hbm.at[pl.ds(starts[k], _CHUNK)], semo
            ).wait()

    @functools.partial(
        pl.kernel,
        mesh=mesh,
        compiler_params=params,
        out_type=jax.ShapeDtypeStruct((_BATCH, _D), jnp.float32),
        scratch_types=[
            pltpu.VMEM((bpw, _SEQ), jnp.int32),
            pltpu.VMEM((_SEQ, _D), jnp.bfloat16),
            pltpu.VMEM((_SEQ, _D), jnp.bfloat16),
            pltpu.VMEM((_SEQ, _D), jnp.bfloat16),
            pltpu.VMEM((_SEQ, _D), jnp.bfloat16),
            pltpu.VMEM((bpw, _D), jnp.float32),
            pltpu.SemaphoreType.DMA,
            pltpu.SemaphoreType.DMA,
            pltpu.SemaphoreType.DMA,
            pltpu.SemaphoreType.DMA,
        ],
    )
    def pool(
        ids_hbm, table_hbm, x_hbm, idx_v, r0_v, r1_v, r2_v, r3_v, out_v, s0, s1, s2, s3
    ):
        wid = lax.axis_index("s") * nc + lax.axis_index("c")
        pltpu.sync_copy(ids_hbm.at[pl.ds(wid * bpw, bpw)], idx_v)
        scale = jnp.float32(1.0 / _SEQ)
        c0, c1 = 104, _SEQ - 104

        def start(buf, sem, row):
            pltpu.async_copy(
                table_hbm.at[idx_v.at[row, pl.ds(0, c0)]], buf.at[pl.ds(0, c0)], sem
            )
            pltpu.async_copy(
                table_hbm.at[idx_v.at[row, pl.ds(c0, c1)]], buf.at[pl.ds(c0, c1)], sem
            )

        def wait(buf, sem, row):
            pltpu.make_async_copy(
                table_hbm.at[idx_v.at[row, pl.ds(0, c0)]], buf.at[pl.ds(0, c0)], sem
            ).wait()
            pltpu.make_async_copy(
                table_hbm.at[idx_v.at[row, pl.ds(c0, c1)]], buf.at[pl.ds(c0, c1)], sem
            ).wait()

        def accum(buf, row):
            def tbody(t, accs):
                new = []
                for c in range(2):
                    lo = buf[t, pl.ds(32 * c, 32)]
                    hi = buf[t + 100, pl.ds(32 * c, 32)]
                    alo, blo = plsc.unpack(lo, format=plsc.PackFormat.INTERLEAVED)
                    ahi, bhi = plsc.unpack(hi, format=plsc.PackFormat.INTERLEAVED)
                    new.append((accs[c][0] + alo + ahi, accs[c][1] + blo + bhi))
                return tuple(new)

            zero = jnp.zeros((16,), jnp.float32)
            accs = lax.fori_loop(
                0, _SEQ // 2, tbody, ((zero, zero), (zero, zero)), unroll=4
            )
            for c in range(2):
                for h in range(2):
                    out_v[row, pl.ds(32 * c + 16 * h, 16)] = accs[c][h] * scale

        bufs = (r0_v, r1_v, r2_v, r3_v)
        sems = (s0, s1, s2, s3)
        for p in range(3):
            start(bufs[p], sems[p], p)

        def body(q, carry):
            for ph in range(4):
                r = 4 * q + ph
                pf = (ph + 3) % 4
                start(bufs[pf], sems[pf], jnp.minimum(r + 3, bpw - 1))
                wait(bufs[ph], sems[ph], r)
                accum(bufs[ph], r)
            return carry

        lax.fori_loop(0, bpw // 4, body, 0)
        for p in range(3):
            wait(bufs[p], sems[p], bpw - 1)
        pltpu.sync_copy(out_v, x_hbm.at[pl.ds(wid * bpw, bpw)])

    return convert, pool


def _linear_body(x_ref, w_ref, b_ref, o_ref):
    o_ref[...] = (
        jnp.dot(x_ref[...], w_ref[...], preferred_element_type=jnp.float32)
        + b_ref[...]
    )


def _linear(x, w, b):
    return pl.pallas_call(
        _linear_body,
        out_shape=jax.ShapeDtypeStruct((_BATCH, _NCLS), jnp.float32),
    )(x, w, b.reshape(1, _NCLS))


def kernel(input_ids, embedding, W, b):
    convert, pool = _build()
    table_bf16 = convert(embedding)
    x = pool(input_ids.astype(jnp.int32), table_bf16)
    logits = _linear(x, W, b)
    return (logits, x)


# speedup vs baseline: 1.3836x; 1.0079x over previous
import functools

import jax
import jax.numpy as jnp
from jax import lax
from jax.experimental import pallas as pl
from jax.experimental.pallas import tpu as pltpu
from jax.experimental.pallas import tpu_sc as plsc

_BATCH = 4096
_SEQ = 200
_D = 64
_NCLS = 2
_VOCAB = 50257
_CHUNK = 400


@functools.cache
def _build():
    info = plsc.get_sparse_core_info()
    nc, ns = info.num_cores, info.num_subcores
    nw = nc * ns
    bpw = _BATCH // nw
    span = -(-_VOCAB // nw)
    nchunk = -(-span // _CHUNK)
    mesh = plsc.VectorSubcoreMesh(core_axis_name="c", subcore_axis_name="s")
    params = pltpu.CompilerParams(
        use_tc_tiling_on_sc=False, needs_layout_passes=False
    )

    @functools.partial(
        pl.kernel,
        mesh=mesh,
        compiler_params=params,
        out_type=jax.ShapeDtypeStruct((nw * span, _D), jnp.bfloat16),
        scratch_types=[
            pltpu.VMEM((_CHUNK, _D), jnp.float32),
            pltpu.VMEM((_CHUNK, _D), jnp.float32),
            pltpu.VMEM((_CHUNK, _D), jnp.bfloat16),
            pltpu.VMEM((_CHUNK, _D), jnp.bfloat16),
            pltpu.SemaphoreType.DMA,
            pltpu.SemaphoreType.DMA,
        ],
    )
    def convert(table_hbm, out_hbm, in_v0, in_v1, out_v0, out_v1, semi, semo):
        wid = lax.axis_index("s") * nc + lax.axis_index("c")
        sw = wid * span
        starts = [jnp.minimum(sw + k * _CHUNK, _VOCAB - _CHUNK) for k in range(nchunk)]
        inb, outb = [in_v0, in_v1], [out_v0, out_v1]

        pltpu.async_copy(table_hbm.at[pl.ds(starts[0], _CHUNK)], inb[0], semi)
        for k in range(nchunk):
            b = k % 2
            pltpu.make_async_copy(
                table_hbm.at[pl.ds(starts[k], _CHUNK)], inb[b], semi
            ).wait()
            if k + 1 < nchunk:
                pltpu.async_copy(
                    table_hbm.at[pl.ds(starts[k + 1], _CHUNK)], inb[1 - b], semi
                )
            if k >= 2:
                pltpu.make_async_copy(
                    outb[b], out_hbm.at[pl.ds(starts[k - 2], _CHUNK)], semo
                ).wait()

            def row(r, carry, b=b):
                for c in range(2):
                    a = inb[b][r, pl.ds(32 * c, 16)]
                    z = inb[b][r, pl.ds(32 * c + 16, 16)]
                    outb[b][r, pl.ds(32 * c, 32)] = plsc.pack(
                        a, z, format=plsc.PackFormat.INTERLEAVED
                    )
                return carry

            lax.fori_loop(0, _CHUNK, row, 0, unroll=4)
            pltpu.async_copy(outb[b], out_hbm.at[pl.ds(starts[k], _CHUNK)], semo)
        for k in (nchunk - 2, nchunk - 1):
            pltpu.make_async_copy(
                outb[k % 2], out_hbm.at[pl.ds(starts[k], _CHUNK)], semo
            ).wait()

    @functools.partial(
        pl.kernel,
        mesh=mesh,
        compiler_params=params,
        out_type=jax.ShapeDtypeStruct((_BATCH, _D), jnp.float32),
        scratch_types=[
            pltpu.VMEM((bpw * _SEQ,), jnp.int32),
            pltpu.VMEM((_SEQ, _D), jnp.bfloat16),
            pltpu.VMEM((_SEQ, _D), jnp.bfloat16),
            pltpu.VMEM((_SEQ, _D), jnp.bfloat16),
            pltpu.VMEM((_SEQ, _D), jnp.bfloat16),
            pltpu.VMEM((_SEQ, _D), jnp.bfloat16),
            pltpu.VMEM((_SEQ, _D), jnp.bfloat16),
            pltpu.VMEM((_SEQ, _D), jnp.bfloat16),
            pltpu.VMEM((_SEQ, _D), jnp.bfloat16),
            pltpu.VMEM((bpw, _D), jnp.float32),
        ]
        + [pltpu.SemaphoreType.DMA] * 8,
    )
    def pool(ids_hbm, table_hbm, x_hbm, idx_v, *rest):
        bufs, (out_v,), sems = rest[:8], rest[8:9], rest[9:]
        wid = lax.axis_index("s") * nc + lax.axis_index("c")
        pltpu.sync_copy(ids_hbm.at[pl.ds(wid * bpw * _SEQ, bpw * _SEQ)], idx_v)
        scale = jnp.float32(1.0 / _SEQ)
        c0, c1 = 104, _SEQ - 104

        def start(buf, sem, row):
            base = pl.multiple_of(row * _SEQ, 8)
            pltpu.async_copy(
                table_hbm.at[idx_v.at[pl.ds(base, c0)]], buf.at[pl.ds(0, c0)], sem
            )
            pltpu.async_copy(
                table_hbm.at[idx_v.at[pl.ds(base + c0, c1)]],
                buf.at[pl.ds(c0, c1)],
                sem,
            )

        def wait(buf, sem, row):
            base = pl.multiple_of(row * _SEQ, 8)
            pltpu.make_async_copy(
                table_hbm.at[idx_v.at[pl.ds(base, c0)]], buf.at[pl.ds(0, c0)], sem
            ).wait()
            pltpu.make_async_copy(
                table_hbm.at[idx_v.at[pl.ds(base + c0, c1)]],
                buf.at[pl.ds(c0, c1)],
                sem,
            ).wait()

        def accum(buf, row):
            def tbody(t, accs):
                new = []
                for c in range(2):
                    lo = buf[t, pl.ds(32 * c, 32)]
                    hi = buf[t + 100, pl.ds(32 * c, 32)]
                    alo, blo = plsc.unpack(lo, format=plsc.PackFormat.INTERLEAVED)
                    ahi, bhi = plsc.unpack(hi, format=plsc.PackFormat.INTERLEAVED)
                    new.append((accs[c][0] + alo + ahi, accs[c][1] + blo + bhi))
                return tuple(new)

            zero = jnp.zeros((16,), jnp.float32)
            accs = lax.fori_loop(
                0, _SEQ // 2, tbody, ((zero, zero), (zero, zero)), unroll=4
            )
            for c in range(2):
                for h in range(2):
                    out_v[row, pl.ds(32 * c + 16 * h, 16)] = accs[c][h] * scale

        nb = 8
        for p in range(nb - 1):
            start(bufs[p], sems[p], p)

        def body(q, carry):
            for ph in range(nb):
                r = nb * q + ph
                pf = (ph + nb - 1) % nb
                start(bufs[pf], sems[pf], jnp.minimum(r + nb - 1, bpw - 1))
                wait(bufs[ph], sems[ph], r)
                accum(bufs[ph], r)
            return carry

        lax.fori_loop(0, bpw // nb, body, 0)
        for p in range(nb - 1):
            wait(bufs[p], sems[p], bpw - 1)
        pltpu.sync_copy(out_v, x_hbm.at[pl.ds(wid * bpw, bpw)])

    return convert, pool


def _linear_body(x_ref, w_ref, b_ref, o_ref):
    o_ref[...] = (
        jnp.dot(x_ref[...], w_ref[...], preferred_element_type=jnp.float32)
        + b_ref[...]
    )


def _linear(x, w, b):
    return pl.pallas_call(
        _linear_body,
        out_shape=jax.ShapeDtypeStruct((_BATCH, _NCLS), jnp.float32),
    )(x, w, b.reshape(1, _NCLS))


def kernel(input_ids, embedding, W, b):
    convert, pool = _build()
    table_bf16 = convert(embedding)
    x = pool(input_ids.astype(jnp.int32).reshape(-1), table_bf16)
    logits = _linear(x, W, b)
    return (logits, x)
